# baseline (speedup 1.0000x reference)
import functools

import jax
import jax.numpy as jnp
from jax import lax
from jax.experimental import pallas as pl
from jax.experimental.pallas import tpu as pltpu
from jax.experimental.pallas import tpu_sc as plsc

D = 64
CTX = 10
NNEG = 20

NC = 2
NS = 16
NW = NC * NS
CHUNK = 512


def _pipelined_gather(tab_h, idx_h, n_chunks, base, out_h, ibufs, bufs, sems):
  i0, i1 = ibufs
  buf0, buf1 = bufs
  sem0, sem1 = sems

  def load_idx(c, ibuf):
    b = pl.multiple_of(base + c * CHUNK, CHUNK)
    pltpu.sync_copy(idx_h.at[pl.ds(b, CHUNK)], ibuf)

  def start(ibuf, buf, sem):
    pltpu.async_copy(tab_h.at[ibuf], buf, sem)

  def wait(ibuf, buf, sem):
    pltpu.make_async_copy(tab_h.at[ibuf], buf, sem).wait()

  def scatter(c, buf):
    b = pl.multiple_of(base + c * CHUNK, CHUNK)
    pltpu.sync_copy(buf, out_h.at[pl.ds(b, CHUNK)])

  if n_chunks == 1:
    load_idx(0, i0)
    start(i0, buf0, sem0)
    wait(i0, buf0, sem0)
    scatter(0, buf0)
    return

  nh = n_chunks // 2
  load_idx(0, i0)
  start(i0, buf0, sem0)

  def body(j, carry):
    c0 = j * 2
    load_idx(c0 + 1, i1)
    start(i1, buf1, sem1)
    wait(i0, buf0, sem0)
    scatter(c0, buf0)

    @pl.when(j < nh - 1)
    def _():
      load_idx(c0 + 2, i0)
      start(i0, buf0, sem0)

    wait(i1, buf1, sem1)
    scatter(c0 + 1, buf1)
    return carry

  lax.fori_loop(0, nh, body, 0)


def _sc_gather(i_emb, o_emb, ctx_idx, tgt_idx, neg_idx):
  nc_ctx = ctx_idx.shape[0] // (NW * CHUNK)
  nc_tgt = tgt_idx.shape[0] // (NW * CHUNK)
  nc_neg = neg_idx.shape[0] // (NW * CHUNK)

  mesh = plsc.VectorSubcoreMesh(core_axis_name="c", subcore_axis_name="s")

  @functools.partial(
      pl.kernel,
      mesh=mesh,
      out_type=[
          jax.ShapeDtypeStruct((NW * nc_ctx * CHUNK, D), jnp.float32),
          jax.ShapeDtypeStruct((NW * nc_tgt * CHUNK, D), jnp.float32),
          jax.ShapeDtypeStruct((NW * nc_neg * CHUNK, D), jnp.float32),
      ],
      scratch_types=[
          pltpu.VMEM((CHUNK,), jnp.int32),
          pltpu.VMEM((CHUNK,), jnp.int32),
          pltpu.VMEM((CHUNK, D), jnp.float32),
          pltpu.VMEM((CHUNK, D), jnp.float32),
          pltpu.SemaphoreType.DMA,
          pltpu.SemaphoreType.DMA,
      ],
      compiler_params=pltpu.CompilerParams(use_tc_tiling_on_sc=False),
  )
  def gather_k(i_emb_h, o_emb_h, ctx_idx_h, tgt_idx_h, neg_idx_h,
               ctx_out, tgt_out, neg_out,
               i0, i1, buf0, buf1, sem0, sem1):
    wid = lax.axis_index("s") * NC + lax.axis_index("c")
    ibufs = (i0, i1)
    bufs = (buf0, buf1)
    sems = (sem0, sem1)
    _pipelined_gather(i_emb_h, ctx_idx_h, nc_ctx, wid * (nc_ctx * CHUNK),
                      ctx_out, ibufs, bufs, sems)
    _pipelined_gather(o_emb_h, tgt_idx_h, nc_tgt, wid * (nc_tgt * CHUNK),
                      tgt_out, ibufs, bufs, sems)
    _pipelined_gather(o_emb_h, neg_idx_h, nc_neg, wid * (nc_neg * CHUNK),
                      neg_out, ibufs, bufs, sems)

  return gather_k(i_emb, o_emb, ctx_idx, tgt_idx, neg_idx)


def _dense_body(ctx_ref, tgt_ref, neg_ref, W1_ref, b1_ref, W2_ref, b2_ref,
                out_ref):
  hp = jnp.dot(ctx_ref[0], W1_ref[pl.ds(0, D), :],
               preferred_element_type=jnp.float32)
  for j in range(1, CTX):
    hp = hp + jnp.dot(ctx_ref[j], W1_ref[pl.ds(j * D, D), :],
                      preferred_element_type=jnp.float32)
  h = jnp.tanh(hp + b1_ref[...])
  logits = jnp.dot(h, W2_ref[...],
                   preferred_element_type=jnp.float32) + b2_ref[...]
  a = jax.nn.softmax(logits, axis=-1)

  attn = a[:, 0:1] * ctx_ref[0]
  for j in range(1, CTX):
    attn = attn + a[:, j:j + 1] * ctx_ref[j]

  pos_dot = jnp.sum(tgt_ref[...] * attn, axis=1)
  acc = jnp.sum(jnp.log(jax.nn.sigmoid(pos_dot)))

  for j in range(NNEG):
    nd = jnp.sum(neg_ref[j] * attn, axis=1)
    acc = acc + jnp.sum(jnp.log(jax.nn.sigmoid(-nd)))

  @pl.when(pl.program_id(0) == 0)
  def _():
    out_ref[0, 0] = 0.0

  out_ref[0, 0] += acc


def kernel(target_wids, context_wids, neg_wids, i_emb, o_emb, W1, b1, W2, b2):
  B = target_wids.shape[0]
  ctx_ids = context_wids.astype(jnp.int32).T.reshape(-1)
  tgt_ids = target_wids.astype(jnp.int32)
  neg_ids = neg_wids.astype(jnp.int32).T.reshape(-1)

  ctx_rows, tgt, neg_rows = _sc_gather(i_emb, o_emb, ctx_ids, tgt_ids,
                                       neg_ids)
  ctx3 = ctx_rows.reshape(CTX, B, D)
  neg3 = neg_rows.reshape(NNEG, B, D)

  bB = 1024
  grid = B // bB
  loss = pl.pallas_call(
      _dense_body,
      grid=(grid,),
      in_specs=[
          pl.BlockSpec((CTX, bB, D), lambda i: (0, i, 0)),
          pl.BlockSpec((bB, D), lambda i: (i, 0)),
          pl.BlockSpec((NNEG, bB, D), lambda i: (0, i, 0)),
          pl.BlockSpec((CTX * D, 50), lambda i: (0, 0)),
          pl.BlockSpec((1, 50), lambda i: (0, 0)),
          pl.BlockSpec((50, CTX), lambda i: (0, 0)),
          pl.BlockSpec((1, CTX), lambda i: (0, 0)),
      ],
      out_specs=pl.BlockSpec((1, 1), lambda i: (0, 0),
                             memory_space=pltpu.SMEM),
      out_shape=jax.ShapeDtypeStruct((1, 1), jnp.float32),
  )(ctx3, tgt, neg3, W1, b1.reshape(1, 50), W2, b2.reshape(1, CTX))

  return -loss[0, 0]


# speedup vs baseline: 1.0463x; 1.0463x over previous
import functools

import jax
import jax.numpy as jnp
from jax import lax
from jax.experimental import pallas as pl
from jax.experimental.pallas import tpu as pltpu
from jax.experimental.pallas import tpu_sc as plsc

D = 64
CTX = 10
NNEG = 20

NC = 2
NS = 16
NW = NC * NS
CHUNK = 512


def _pipelined_gather(tab_h, idx_h, n_slots, out_h, spw, wid, ibufs, bufs,
                      sems):
  i0, i1 = ibufs
  buf0, buf1 = bufs
  sem0, sem1 = sems
  B = out_h.shape[0]
  base = wid * spw

  def load_idx(j, ibuf):
    b = pl.multiple_of(j * B + base, CHUNK)
    pltpu.sync_copy(idx_h.at[pl.ds(b, CHUNK)], ibuf)

  def start(ibuf, buf, sem):
    pltpu.async_copy(tab_h.at[ibuf], buf, sem)

  def wait(ibuf, buf, sem):
    pltpu.make_async_copy(tab_h.at[ibuf], buf, sem).wait()

  def scatter(j, buf):
    pltpu.sync_copy(buf, out_h.at[pl.ds(base, CHUNK),
                                  pl.ds(pl.multiple_of(j * D, D), D)])

  if n_slots == 1:
    load_idx(0, i0)
    start(i0, buf0, sem0)
    wait(i0, buf0, sem0)
    pltpu.sync_copy(buf0, out_h.at[pl.ds(base, CHUNK)])
    return

  nh = n_slots // 2
  load_idx(0, i0)
  start(i0, buf0, sem0)

  def body(k, carry):
    j0 = k * 2
    load_idx(j0 + 1, i1)
    start(i1, buf1, sem1)
    wait(i0, buf0, sem0)
    scatter(j0, buf0)

    @pl.when(k < nh - 1)
    def _():
      load_idx(j0 + 2, i0)
      start(i0, buf0, sem0)

    wait(i1, buf1, sem1)
    scatter(j0 + 1, buf1)
    return carry

  lax.fori_loop(0, nh, body, 0)


def _sc_gather(i_emb, o_emb, ctx_idx, tgt_idx, neg_idx):
  B = tgt_idx.shape[0]

  mesh = plsc.VectorSubcoreMesh(core_axis_name="c", subcore_axis_name="s")

  @functools.partial(
      pl.kernel,
      mesh=mesh,
      out_type=[
          jax.ShapeDtypeStruct((B, CTX * D), jnp.float32),
          jax.ShapeDtypeStruct((B, D), jnp.float32),
          jax.ShapeDtypeStruct((B, NNEG * D), jnp.float32),
      ],
      scratch_types=[
          pltpu.VMEM((CHUNK,), jnp.int32),
          pltpu.VMEM((CHUNK,), jnp.int32),
          pltpu.VMEM((CHUNK, D), jnp.float32),
          pltpu.VMEM((CHUNK, D), jnp.float32),
          pltpu.SemaphoreType.DMA,
          pltpu.SemaphoreType.DMA,
      ],
      compiler_params=pltpu.CompilerParams(use_tc_tiling_on_sc=False),
  )
  def gather_k(i_emb_h, o_emb_h, ctx_idx_h, tgt_idx_h, neg_idx_h,
               ctx_out, tgt_out, neg_out,
               i0, i1, buf0, buf1, sem0, sem1):
    wid = lax.axis_index("s") * NC + lax.axis_index("c")
    ibufs = (i0, i1)
    bufs = (buf0, buf1)
    sems = (sem0, sem1)
    spw = B // NW
    _pipelined_gather(i_emb_h, ctx_idx_h, CTX, ctx_out, spw, wid, ibufs,
                      bufs, sems)
    _pipelined_gather(o_emb_h, tgt_idx_h, 1, tgt_out, spw, wid, ibufs,
                      bufs, sems)
    _pipelined_gather(o_emb_h, neg_idx_h, NNEG, neg_out, spw, wid, ibufs,
                      bufs, sems)

  return gather_k(i_emb, o_emb, ctx_idx, tgt_idx, neg_idx)


def _dense_body(ctx_ref, tgt_ref, neg_ref, W1_ref, b1_ref, W2_ref, b2_ref,
                out_ref):
  ctx = ctx_ref[...]
  h = jnp.tanh(
      jnp.dot(ctx, W1_ref[...], preferred_element_type=jnp.float32)
      + b1_ref[...])
  logits = jnp.dot(h, W2_ref[...],
                   preferred_element_type=jnp.float32) + b2_ref[...]
  a = jax.nn.softmax(logits, axis=-1)

  attn = a[:, 0:1] * ctx[:, 0:D]
  for j in range(1, CTX):
    attn = attn + a[:, j:j + 1] * ctx[:, j * D:(j + 1) * D]

  pos_dot = jnp.sum(tgt_ref[...] * attn, axis=1)
  acc = jnp.sum(jnp.log(jax.nn.sigmoid(pos_dot)))

  neg = neg_ref[...]
  for j in range(NNEG):
    nd = jnp.sum(neg[:, j * D:(j + 1) * D] * attn, axis=1)
    acc = acc + jnp.sum(jnp.log(jax.nn.sigmoid(-nd)))

  @pl.when(pl.program_id(0) == 0)
  def _():
    out_ref[0, 0] = 0.0

  out_ref[0, 0] += acc


def kernel(target_wids, context_wids, neg_wids, i_emb, o_emb, W1, b1, W2, b2):
  B = target_wids.shape[0]
  ctx_ids = context_wids.astype(jnp.int32).T.reshape(-1)
  tgt_ids = target_wids.astype(jnp.int32)
  neg_ids = neg_wids.astype(jnp.int32).T.reshape(-1)

  ctx_flat, tgt, neg_flat = _sc_gather(i_emb, o_emb, ctx_ids, tgt_ids,
                                       neg_ids)

  bB = 1024
  grid = B // bB
  loss = pl.pallas_call(
      _dense_body,
      grid=(grid,),
      in_specs=[
          pl.BlockSpec((bB, CTX * D), lambda i: (i, 0)),
          pl.BlockSpec((bB, D), lambda i: (i, 0)),
          pl.BlockSpec((bB, NNEG * D), lambda i: (i, 0)),
          pl.BlockSpec((CTX * D, 50), lambda i: (0, 0)),
          pl.BlockSpec((1, 50), lambda i: (0, 0)),
          pl.BlockSpec((50, CTX), lambda i: (0, 0)),
          pl.BlockSpec((1, CTX), lambda i: (0, 0)),
      ],
      out_specs=pl.BlockSpec((1, 1), lambda i: (0, 0),
                             memory_space=pltpu.SMEM),
      out_shape=jax.ShapeDtypeStruct((1, 1), jnp.float32),
  )(ctx_flat, tgt, neg_flat, W1, b1.reshape(1, 50), W2, b2.reshape(1, CTX))

  return -loss[0, 0]


# speedup vs baseline: 1.4448x; 1.3809x over previous
import functools

import jax
import jax.numpy as jnp
from jax import lax
from jax.experimental import pallas as pl
from jax.experimental.pallas import tpu as pltpu
from jax.experimental.pallas import tpu_sc as plsc

D = 64
CTX = 10
NNEG = 20

NC = 2
NS = 16
NW = NC * NS
L = 16


def _sc_gather(i_emb, o_emb, ctx_ids, tgt_ids, neg_ids):
  B = tgt_ids.shape[0]
  spw = B // NW

  mesh = plsc.VectorSubcoreMesh(core_axis_name="c", subcore_axis_name="s")

  @functools.partial(
      pl.kernel,
      mesh=mesh,
      out_type=[
          jax.ShapeDtypeStruct((CTX // 2 * B, 2 * D), jnp.float32),
          jax.ShapeDtypeStruct((B, D), jnp.float32),
          jax.ShapeDtypeStruct((NNEG // 2 * B, 2 * D), jnp.float32),
      ],
      scratch_types=[
          pltpu.VMEM((spw,), jnp.int32),
          pltpu.VMEM((spw,), jnp.int32),
          pltpu.VMEM((spw, D), jnp.float32),
          pltpu.VMEM((spw, D), jnp.float32),
          pltpu.SemaphoreType.DMA,
          pltpu.SemaphoreType.DMA,
      ],
      compiler_params=pltpu.CompilerParams(use_tc_tiling_on_sc=False),
  )
  def gather_k(i_emb_h, o_emb_h, ctx_ids_h, tgt_ids_h, neg_ids_h,
               ctx_out, tgt_out, neg_out,
               i0, i1, buf0, buf1, sem0, sem1):
    wid = lax.axis_index("s") * NC + lax.axis_index("c")
    base = wid * spw
    ibufs = (i0, i1)
    bufs = (buf0, buf1)
    sems = (sem0, sem1)

    def build_idx(ids_h, j, ibuf):
      pltpu.sync_copy(ids_h.at[pl.ds(j * B + base, spw)], ibuf)

    def run(tab_h, n_slots, build, scatter):
      def fire(j, slot):
        build(j, ibufs[slot])
        pltpu.async_copy(tab_h.at[ibufs[slot]], bufs[slot], sems[slot])

      def drain(j, slot):
        pltpu.make_async_copy(tab_h.at[ibufs[slot]], bufs[slot],
                              sems[slot]).wait()
        scatter(j, bufs[slot])

      if n_slots == 1:
        fire(0, 0)
        drain(0, 0)
        return

      fire(0, 0)

      def body(k, carry):
        j0 = k * 2
        fire(j0 + 1, 1)
        drain(j0, 0)

        @pl.when(k < n_slots // 2 - 1)
        def _():
          fire(j0 + 2, 0)

        drain(j0 + 1, 1)
        return carry

      lax.fori_loop(0, n_slots // 2, body, 0)

    def scatter_pair(out_h):
      def scatter(j, buf):
        row0 = (j // 2) * B + base
        col0 = (j % 2) * D
        pltpu.sync_copy(buf, out_h.at[pl.ds(row0, spw), pl.ds(col0, D)])
      return scatter

    run(i_emb_h, CTX,
        lambda j, ibuf: build_idx(ctx_ids_h, j, ibuf),
        scatter_pair(ctx_out))
    run(o_emb_h, 1,
        lambda j, ibuf: pltpu.sync_copy(tgt_ids_h.at[pl.ds(base, spw)],
                                        ibuf),
        lambda j, buf: pltpu.sync_copy(buf, tgt_out.at[pl.ds(base, spw)]))
    run(o_emb_h, NNEG,
        lambda j, ibuf: build_idx(neg_ids_h, j, ibuf),
        scatter_pair(neg_out))

  return gather_k(i_emb, o_emb, ctx_ids, tgt_ids, neg_ids)


def _dense_body(ctx_ref, tgt_ref, neg_ref, W1_ref, b1_ref, W2_ref, b2_ref,
                out_ref):
  f32 = jnp.float32

  hp = jnp.dot(ctx_ref[0], W1_ref[pl.ds(0, 2 * D), :],
               preferred_element_type=f32)
  for jp in range(1, CTX // 2):
    hp = hp + jnp.dot(ctx_ref[jp], W1_ref[pl.ds(jp * 2 * D, 2 * D), :],
                      preferred_element_type=f32)
  h = jnp.tanh(hp + b1_ref[...])
  logits = jnp.dot(h, W2_ref[...],
                   preferred_element_type=f32) + b2_ref[...]
  a = jax.nn.softmax(logits, axis=-1)

  attn128 = jnp.zeros(hp.shape[:1] + (2 * D,), f32)
  lane128 = lax.broadcasted_iota(jnp.int32, hp.shape[:1] + (2 * D,), 1)
  for jp in range(CTX // 2):
    aw = jnp.where(lane128 < D, a[:, 2 * jp:2 * jp + 1],
                   a[:, 2 * jp + 1:2 * jp + 2])
    attn128 = attn128 + aw * ctx_ref[jp]
  attn = attn128[:, 0:D] + attn128[:, D:2 * D]

  pos_dot = jnp.sum(tgt_ref[...] * attn, axis=1, keepdims=True)

  cols128 = lax.broadcasted_iota(jnp.int32, (2 * D, 2), 0)
  sel = lax.broadcasted_iota(jnp.int32, (2 * D, 2), 1)
  seg2 = jnp.where(sel == 0, cols128 < D, cols128 >= D).astype(f32)
  attn2 = jnp.concatenate([attn, attn], axis=1)
  nds = []
  for k in range(NNEG // 2):
    prod = neg_ref[k] * attn2
    nds.append(jnp.concatenate(
        [jnp.sum(prod[:, 0:D], axis=1, keepdims=True),
         jnp.sum(prod[:, D:2 * D], axis=1, keepdims=True)], axis=1))
  all_dots = jnp.concatenate([pos_dot] + [-n for n in nds], axis=1)

  acc = jnp.sum(jnp.log(jax.nn.sigmoid(all_dots)))

  @pl.when(pl.program_id(0) == 0)
  def _():
    out_ref[0, 0] = 0.0

  out_ref[0, 0] += acc


def kernel(target_wids, context_wids, neg_wids, i_emb, o_emb, W1, b1, W2, b2):
  B = target_wids.shape[0]
  ctx_ids = context_wids.astype(jnp.int32).T.reshape(-1)
  tgt_ids = target_wids.astype(jnp.int32)
  neg_ids = neg_wids.astype(jnp.int32).T.reshape(-1)

  ctx_pair, tgt, neg_pair = _sc_gather(i_emb, o_emb, ctx_ids, tgt_ids,
                                       neg_ids)
  ctx3 = ctx_pair.reshape(CTX // 2, B, 2 * D)
  neg3 = neg_pair.reshape(NNEG // 2, B, 2 * D)

  bB = 1024
  grid = B // bB
  loss = pl.pallas_call(
      _dense_body,
      grid=(grid,),
      in_specs=[
          pl.BlockSpec((CTX // 2, bB, 2 * D), lambda i: (0, i, 0)),
          pl.BlockSpec((bB, D), lambda i: (i, 0)),
          pl.BlockSpec((NNEG // 2, bB, 2 * D), lambda i: (0, i, 0)),
          pl.BlockSpec((CTX * D, 50), lambda i: (0, 0)),
          pl.BlockSpec((1, 50), lambda i: (0, 0)),
          pl.BlockSpec((50, CTX), lambda i: (0, 0)),
          pl.BlockSpec((1, CTX), lambda i: (0, 0)),
      ],
      out_specs=pl.BlockSpec((1, 1), lambda i: (0, 0),
                             memory_space=pltpu.SMEM),
      out_shape=jax.ShapeDtypeStruct((1, 1), jnp.float32),
  )(ctx3, tgt, neg3, W1, b1.reshape(1, 50), W2, b2.reshape(1, CTX))

  return -loss[0, 0]
